# Initial kernel scaffold; baseline (speedup 1.0000x reference)
import jax
import jax.numpy as jnp
from jax.experimental import pallas as pl


def kernel(frames):
    raise NotImplementedError("write your pallas kernel here")


# speedup vs baseline: 1.8216x; 1.8216x over previous
import numpy as np
import jax
import jax.numpy as jnp
from jax.experimental import pallas as pl

_ALPHA = 4


def _pack_body(frames_ref, slow_ref, fast_ref):
    x = frames_ref[...]
    fast_ref[...] = x
    slow_ref[...] = x


def kernel(frames):
    C, T, H, W = frames.shape
    n_slow = T // _ALPHA
    idx = np.linspace(0.0, T - 1, n_slow).astype(np.int32)
    slot_np = np.array([(t * (n_slow - 1) + T - 2) // (T - 1) for t in range(T)])
    assert all(slot_np[int(i)] == s for s, i in enumerate(idx))
    assert all(int(idx[slot_np[t]]) >= t for t in range(T))

    def in_map(t):
        return (0, t, 0, 0)

    def slow_map(t):
        return (0, (t * (n_slow - 1) + T - 2) // (T - 1), 0, 0)

    slow, fast = pl.pallas_call(
        _pack_body,
        grid=(T,),
        in_specs=[pl.BlockSpec((C, 1, H, W), in_map)],
        out_specs=[
            pl.BlockSpec((C, 1, H, W), slow_map),
            pl.BlockSpec((C, 1, H, W), in_map),
        ],
        out_shape=[
            jax.ShapeDtypeStruct((C, n_slow, H, W), frames.dtype),
            jax.ShapeDtypeStruct((C, T, H, W), frames.dtype),
        ],
    )(frames)
    return (slow, fast)


# speedup vs baseline: 2.1796x; 1.1965x over previous
import numpy as np
import jax
import jax.numpy as jnp
from jax.experimental import pallas as pl

_ALPHA = 4


def _pack_body(frames_ref, slow_ref, fast_ref):
    s = pl.program_id(0)
    n_slow = pl.num_programs(0)
    T = n_slow * _ALPHA
    x = frames_ref[...]
    fast_ref[...] = x
    off = (s * (T - 1)) // (n_slow - 1) - _ALPHA * s
    slow_ref[...] = frames_ref[:, pl.ds(off, 1), :, :]


def kernel(frames):
    C, T, H, W = frames.shape
    n_slow = T // _ALPHA
    idx = np.linspace(0.0, T - 1, n_slow).astype(np.int32)
    assert all(_ALPHA * s <= int(i) < _ALPHA * (s + 1) for s, i in enumerate(idx))
    assert all(int(i) == (s * (T - 1)) // (n_slow - 1) for s, i in enumerate(idx))

    def group_map(s):
        return (0, s, 0, 0)

    slow, fast = pl.pallas_call(
        _pack_body,
        grid=(n_slow,),
        in_specs=[pl.BlockSpec((C, _ALPHA, H, W), group_map)],
        out_specs=[
            pl.BlockSpec((C, 1, H, W), group_map),
            pl.BlockSpec((C, _ALPHA, H, W), group_map),
        ],
        out_shape=[
            jax.ShapeDtypeStruct((C, n_slow, H, W), frames.dtype),
            jax.ShapeDtypeStruct((C, T, H, W), frames.dtype),
        ],
    )(frames)
    return (slow, fast)
